# baseline (speedup 1.0000x reference)
import functools

import jax
import jax.numpy as jnp
from jax import lax
from jax.experimental import pallas as pl
from jax.experimental.pallas import tpu as pltpu
from jax.experimental.pallas import tpu_sc as plsc

NC = 2
NS = 16
NW = NC * NS
L = 16
ND = 6


def _prep_kernel(h, w, params_ref, tbl_ref, x1_ref, x2_ref, qa_ref, lin_ref):
    blk = tbl_ref[...]
    lo = lax.bitcast_convert_type(blk.astype(jnp.bfloat16), jnp.uint16)
    col0_up = jnp.concatenate([lo[1:, :1], lo[:1, :1]], axis=0)
    hi = jnp.concatenate([lo[:, 1:], col0_up], axis=1)
    qa_ref[...] = lax.bitcast_convert_type(
        lo.astype(jnp.uint32) | (hi.astype(jnp.uint32) << 16), jnp.int32)

    sy = params_ref[0]
    sx = params_ref[1]
    oy = params_ref[2]
    ox = params_ref[3]
    qy = jnp.maximum(x1_ref[...] * sy + oy, 0.0)
    qx = jnp.maximum(x2_ref[...] * sx + ox, 0.0)
    fy = jnp.minimum(qy.astype(jnp.int32), h - 2)
    fx = jnp.minimum(qx.astype(jnp.int32), w - 2)
    lin_ref[...] = fy * w + fx


def _prep(h, w, params, t128, x1r, x2r):
    g = 32
    tr = t128.shape[0] // g
    xr = x1r.shape[0] // g
    xspec = pl.BlockSpec((xr, 128), lambda i: (i, 0))
    return pl.pallas_call(
        functools.partial(_prep_kernel, h, w),
        out_shape=(jax.ShapeDtypeStruct(t128.shape, jnp.int32),
                   jax.ShapeDtypeStruct((x1r.shape[0], 128), jnp.int32)),
        grid=(g,),
        in_specs=[pl.BlockSpec(memory_space=pltpu.SMEM),
                  pl.BlockSpec((tr, 128), lambda i: (i, 0)),
                  xspec, xspec],
        out_specs=(pl.BlockSpec((tr, 128), lambda i: (i, 0)), xspec),
    )(params, t128, x1r, x2r)


def _make_sc_gather(n, w):
    per_w = n // NW
    t = 2048
    nb = per_w // t
    mesh = plsc.VectorSubcoreMesh(core_axis_name="c", subcore_axis_name="s")

    ring = lambda shp, dt: [pltpu.VMEM(shp, dt) for _ in range(ND)]

    @functools.partial(
        pl.kernel,
        mesh=mesh,
        out_type=(jax.ShapeDtypeStruct((n,), jnp.int32),
                  jax.ShapeDtypeStruct((n,), jnp.int32)),
        scratch_types=(
            [pltpu.VMEM((per_w,), jnp.int32)]
            + ring((2 * t,), jnp.int32) + ring((2 * t,), jnp.int32)
            + [pltpu.SemaphoreType.DMA] * ND
        ),
    )
    def kern(lin_hbm, qa_hbm, v1_hbm, v2_hbm, lin_v, *sc):
        idxs, valss = sc[0:ND], sc[ND:2 * ND]
        sems = sc[2 * ND:3 * ND]

        cid = lax.axis_index("c")
        sid = lax.axis_index("s")
        wid = sid * NC + cid
        base_w = wid * per_w

        def build_idx(b, p):
            idx_v = idxs[p]
            base = b * t

            def body(j, carry):
                s = j * L
                v = lin_v[pl.ds(base + s, L)]
                idx_v[pl.ds(s, L)] = v
                idx_v[pl.ds(t + s, L)] = v + w
                return carry

            lax.fori_loop(0, t // L, body, 0, unroll=8)

        def start_gather(p):
            return pltpu.async_copy(qa_hbm.at[idxs[p]], valss[p], sems[p])

        def store_vals(b, p):
            off = base_w + b * t
            pltpu.sync_copy(valss[p].at[pl.ds(0, t)], v1_hbm.at[pl.ds(off, t)])
            pltpu.sync_copy(valss[p].at[pl.ds(t, t)], v2_hbm.at[pl.ds(off, t)])

        with jax.named_scope("load_lin"):
            pltpu.sync_copy(lin_hbm.at[pl.ds(base_w, per_w)], lin_v)

        handles = {}
        for b in range(nb):
            p = b % ND
            if b >= ND:
                with jax.named_scope("gather_wait"):
                    handles[b - ND].wait()
                with jax.named_scope("store_vals"):
                    store_vals(b - ND, p)
            with jax.named_scope("build_idx"):
                build_idx(b, p)
            handles[b] = start_gather(p)
        for b in range(nb - ND, nb):
            with jax.named_scope("gather_wait"):
                handles[b].wait()
            with jax.named_scope("store_vals"):
                store_vals(b, b % ND)

    return kern


def _combine_kernel(h, w, params_ref, x1_ref, x2_ref, v1_ref, v2_ref, out_ref):
    sy = params_ref[0]
    sx = params_ref[1]
    oy = params_ref[2]
    ox = params_ref[3]
    qy = jnp.maximum(x1_ref[...] * sy + oy, 0.0)
    qx = jnp.maximum(x2_ref[...] * sx + ox, 0.0)
    fy = jnp.minimum(jnp.floor(qy), float(h - 2))
    fx = jnp.minimum(jnp.floor(qx), float(w - 2))
    ay = jnp.minimum(qy - fy, 1.0)
    ax = jnp.minimum(qx - fx, 1.0)
    v1 = v1_ref[...]
    v2 = v2_ref[...]
    himask = jnp.int32(-65536)
    tl = lax.bitcast_convert_type(v1 << 16, jnp.float32)
    tr = lax.bitcast_convert_type(v1 & himask, jnp.float32)
    bl = lax.bitcast_convert_type(v2 << 16, jnp.float32)
    br = lax.bitcast_convert_type(v2 & himask, jnp.float32)
    top = ax * (tr - tl) + tl
    bot = ax * (br - bl) + bl
    out_ref[...] = ay * (bot - top) + top


def _combine(h, w, params, x1r, x2r, v1r, v2r):
    rows = x1r.shape[0]
    blk = 1024
    spec = pl.BlockSpec((blk, 128), lambda i: (i, 0))
    return pl.pallas_call(
        functools.partial(_combine_kernel, h, w),
        out_shape=jax.ShapeDtypeStruct((rows, 128), jnp.float32),
        grid=(rows // blk,),
        in_specs=[pl.BlockSpec(memory_space=pltpu.SMEM),
                  spec, spec, spec, spec],
        out_specs=spec,
    )(params, x1r, x2r, v1r, v2r)


def kernel(inputs, grid, bounds):
    n = inputs.shape[0]
    _, h, w, _ = grid.shape
    scale = (jnp.array([h, w], jnp.float32) - 1.0) / (bounds[1] - bounds[0])
    off = -bounds[0] * scale
    params = jnp.concatenate([scale, off]).astype(jnp.float32)
    rows = n // 128
    x1r = inputs[:, 0].reshape(rows, 128)
    x2r = inputs[:, 1].reshape(rows, 128)
    qa, lin = _prep(h, w, params, grid.reshape(h * w // 128, 128), x1r, x2r)
    v1, v2 = _make_sc_gather(n, w)(lin.reshape(-1), qa.reshape(-1))
    out = _combine(h, w, params, x1r, x2r,
                   v1.reshape(rows, 128), v2.reshape(rows, 128))
    return out.reshape(n, 1)


# speedup vs baseline: 1.1257x; 1.1257x over previous
import functools

import jax
import jax.numpy as jnp
from jax import lax
from jax.experimental import pallas as pl
from jax.experimental.pallas import tpu as pltpu
from jax.experimental.pallas import tpu_sc as plsc

NC = 2
NS = 16
NW = NC * NS
L = 16
ND = 4


def _prep_kernel(h, w, params_ref, tbl_ref, x1_ref, x2_ref, qa_ref, lin_ref):
    blk = tbl_ref[...]
    lo = lax.bitcast_convert_type(blk.astype(jnp.bfloat16), jnp.uint16)
    col0_up = jnp.concatenate([lo[1:, :1], lo[:1, :1]], axis=0)
    hi = jnp.concatenate([lo[:, 1:], col0_up], axis=1)
    qa_ref[...] = lax.bitcast_convert_type(
        lo.astype(jnp.uint32) | (hi.astype(jnp.uint32) << 16), jnp.int32)

    sy = params_ref[0]
    sx = params_ref[1]
    oy = params_ref[2]
    ox = params_ref[3]
    qy = jnp.maximum(x1_ref[...] * sy + oy, 0.0)
    qx = jnp.maximum(x2_ref[...] * sx + ox, 0.0)
    fy = jnp.minimum(qy.astype(jnp.int32), h - 2)
    fx = jnp.minimum(qx.astype(jnp.int32), w - 2)
    lin_ref[...] = fy * w + fx


def _prep(h, w, params, t128, x1r, x2r):
    g = 8
    tr = t128.shape[0] // g
    xr = x1r.shape[0] // g
    xspec = pl.BlockSpec((xr, 128), lambda i: (i, 0))
    return pl.pallas_call(
        functools.partial(_prep_kernel, h, w),
        out_shape=(jax.ShapeDtypeStruct(t128.shape, jnp.int32),
                   jax.ShapeDtypeStruct((x1r.shape[0], 128), jnp.int32)),
        grid=(g,),
        in_specs=[pl.BlockSpec(memory_space=pltpu.SMEM),
                  pl.BlockSpec((tr, 128), lambda i: (i, 0)),
                  xspec, xspec],
        out_specs=(pl.BlockSpec((tr, 128), lambda i: (i, 0)), xspec),
    )(params, t128, x1r, x2r)


def _make_sc_gather(n, w):
    per_w = n // NW
    t = 2048
    nb = per_w // t
    mesh = plsc.VectorSubcoreMesh(core_axis_name="c", subcore_axis_name="s")

    ring = lambda shp, dt: [pltpu.VMEM(shp, dt) for _ in range(ND)]

    @functools.partial(
        pl.kernel,
        mesh=mesh,
        out_type=(jax.ShapeDtypeStruct((n,), jnp.int32),
                  jax.ShapeDtypeStruct((n,), jnp.int32)),
        scratch_types=(
            [pltpu.VMEM((per_w,), jnp.int32)]
            + ring((2 * t,), jnp.int32) + ring((2 * t,), jnp.int32)
            + [pltpu.SemaphoreType.DMA] * ND
        ),
    )
    def kern(lin_hbm, qa_hbm, v1_hbm, v2_hbm, lin_v, *sc):
        idxs, valss = sc[0:ND], sc[ND:2 * ND]
        sems = sc[2 * ND:3 * ND]

        cid = lax.axis_index("c")
        sid = lax.axis_index("s")
        wid = sid * NC + cid
        base_w = wid * per_w

        def build_idx(b, p):
            idx_v = idxs[p]
            base = b * t

            def body(j, carry):
                s = j * L
                v = lin_v[pl.ds(base + s, L)]
                idx_v[pl.ds(s, L)] = v
                idx_v[pl.ds(t + s, L)] = v + w
                return carry

            lax.fori_loop(0, t // L, body, 0, unroll=8)

        def start_gather(p):
            return pltpu.async_copy(qa_hbm.at[idxs[p]], valss[p], sems[p])

        def store_vals(b, p):
            off = base_w + b * t
            pltpu.sync_copy(valss[p].at[pl.ds(0, t)], v1_hbm.at[pl.ds(off, t)])
            pltpu.sync_copy(valss[p].at[pl.ds(t, t)], v2_hbm.at[pl.ds(off, t)])

        with jax.named_scope("load_lin"):
            pltpu.sync_copy(lin_hbm.at[pl.ds(base_w, per_w)], lin_v)

        handles = {}
        for b in range(nb):
            p = b % ND
            if b >= ND:
                with jax.named_scope("gather_wait"):
                    handles[b - ND].wait()
                with jax.named_scope("store_vals"):
                    store_vals(b - ND, p)
            with jax.named_scope("build_idx"):
                build_idx(b, p)
            handles[b] = start_gather(p)
        for b in range(nb - ND, nb):
            with jax.named_scope("gather_wait"):
                handles[b].wait()
            with jax.named_scope("store_vals"):
                store_vals(b, b % ND)

    return kern


def _combine_kernel(h, w, params_ref, x1_ref, x2_ref, v1_ref, v2_ref, out_ref):
    sy = params_ref[0]
    sx = params_ref[1]
    oy = params_ref[2]
    ox = params_ref[3]
    qy = jnp.maximum(x1_ref[...] * sy + oy, 0.0)
    qx = jnp.maximum(x2_ref[...] * sx + ox, 0.0)
    fy = jnp.minimum(jnp.floor(qy), float(h - 2))
    fx = jnp.minimum(jnp.floor(qx), float(w - 2))
    ay = jnp.minimum(qy - fy, 1.0)
    ax = jnp.minimum(qx - fx, 1.0)
    v1 = v1_ref[...]
    v2 = v2_ref[...]
    himask = jnp.int32(-65536)
    tl = lax.bitcast_convert_type(v1 << 16, jnp.float32)
    tr = lax.bitcast_convert_type(v1 & himask, jnp.float32)
    bl = lax.bitcast_convert_type(v2 << 16, jnp.float32)
    br = lax.bitcast_convert_type(v2 & himask, jnp.float32)
    top = ax * (tr - tl) + tl
    bot = ax * (br - bl) + bl
    out_ref[...] = ay * (bot - top) + top


def _combine(h, w, params, x1r, x2r, v1r, v2r):
    rows = x1r.shape[0]
    blk = 1024
    spec = pl.BlockSpec((blk, 128), lambda i: (i, 0))
    return pl.pallas_call(
        functools.partial(_combine_kernel, h, w),
        out_shape=jax.ShapeDtypeStruct((rows, 128), jnp.float32),
        grid=(rows // blk,),
        in_specs=[pl.BlockSpec(memory_space=pltpu.SMEM),
                  spec, spec, spec, spec],
        out_specs=spec,
    )(params, x1r, x2r, v1r, v2r)


def kernel(inputs, grid, bounds):
    n = inputs.shape[0]
    _, h, w, _ = grid.shape
    scale = (jnp.array([h, w], jnp.float32) - 1.0) / (bounds[1] - bounds[0])
    off = -bounds[0] * scale
    params = jnp.concatenate([scale, off]).astype(jnp.float32)
    rows = n // 128
    x1r = inputs[:, 0].reshape(rows, 128)
    x2r = inputs[:, 1].reshape(rows, 128)
    qa, lin = _prep(h, w, params, grid.reshape(h * w // 128, 128), x1r, x2r)
    v1, v2 = _make_sc_gather(n, w)(lin.reshape(-1), qa.reshape(-1))
    out = _combine(h, w, params, x1r, x2r,
                   v1.reshape(rows, 128), v2.reshape(rows, 128))
    return out.reshape(n, 1)
